# baseline (speedup 1.0000x reference)
import functools
import jax
import jax.numpy as jnp
from jax import lax
from jax.experimental import pallas as pl
from jax.experimental.pallas import tpu as pltpu
from jax.experimental.pallas import tpu_sc as plsc

N_NODES = 10000
D = 128
NC = 2
NS = 16
CH = 128
SPC = 8
NPAD = 10112
RPT = NPAD // NS
CROW = 80
SPLIT = (48, 112)


def _run_stage_loop(nchunk, x_hbm, src_hbm, dst_hbm, acc_sh,
                    src_v, dst_v, rows, sg, ss, s):
    def stage(st, _):
        pltpu.sync_copy(src_hbm.at[s, pl.ds(st * SPC, SPC)], src_v)
        pltpu.sync_copy(dst_hbm.at[s, pl.ds(st * SPC, SPC)], dst_v)
        pltpu.async_copy(x_hbm.at[src_v.at[0]], rows[0], sg[0])
        for j in range(SPC):
            b = j & 1
            if j + 1 < SPC:
                if j >= 1:
                    pltpu.make_async_copy(
                        rows[1 - b], acc_sh.at[dst_v.at[j - 1]],
                        ss[1 - b]).wait()
                pltpu.async_copy(x_hbm.at[src_v.at[j + 1]],
                                 rows[1 - b], sg[1 - b])
            pltpu.make_async_copy(x_hbm.at[src_v.at[j]], rows[b],
                                  sg[b]).wait()
            pltpu.async_copy(rows[b], acc_sh.at[dst_v.at[j]], ss[b],
                             add=True)
        for b in range(2):
            pltpu.make_async_copy(
                rows[b], acc_sh.at[dst_v.at[SPC - 2 + b]], ss[b]).wait()
        return 0
    lax.fori_loop(0, nchunk // SPC, stage, 0)


def _sc_agg_body(na, nb, x_hbm, src0_hbm, dst0_hbm, src1_hbm, dst1_hbm,
                 z_hbm, out_hbm, src_v, dst_v, rows0_v, rows1_v,
                 acc_sh, sg0, sg1, ss0, ss1):
    c = lax.axis_index("c")
    s = lax.axis_index("s")

    nfull = RPT // CH
    rem = RPT % CH
    pltpu.sync_copy(z_hbm, rows0_v)
    for k in range(nfull):
        pltpu.sync_copy(rows0_v, acc_sh.at[pl.ds(s * RPT + k * CH, CH)])
    if rem:
        pltpu.sync_copy(rows0_v.at[pl.ds(0, rem)],
                        acc_sh.at[pl.ds(s * RPT + nfull * CH, rem)])

    plsc.subcore_barrier()

    rows = (rows0_v, rows1_v)
    sg = (sg0, sg1)
    ss = (ss0, ss1)

    @pl.when(c == 0)
    def _():
        _run_stage_loop(na, x_hbm, src0_hbm, dst0_hbm, acc_sh,
                        src_v, dst_v, rows, sg, ss, s)

    @pl.when(c == 1)
    def _():
        _run_stage_loop(nb, x_hbm, src1_hbm, dst1_hbm, acc_sh,
                        src_v, dst_v, rows, sg, ss, s)

    plsc.subcore_barrier()

    for k in range(nfull + (1 if rem else 0)):
        w = CH if k < nfull else rem
        base = s * RPT + k * CH
        pltpu.sync_copy(acc_sh.at[pl.ds(base, w)], rows0_v.at[pl.ds(0, w)])
        pltpu.sync_copy(rows0_v.at[pl.ds(0, w)],
                        out_hbm.at[c, pl.ds(base, w)])


def _make_sc_agg(na, nb):
    mesh = plsc.VectorSubcoreMesh(core_axis_name="c", subcore_axis_name="s")
    return pl.kernel(
        functools.partial(_sc_agg_body, na, nb),
        out_type=[jax.ShapeDtypeStruct((NC, NPAD, D), jnp.float32)],
        mesh=mesh,
        compiler_params=pltpu.CompilerParams(needs_layout_passes=False),
        scratch_types=[
            pltpu.VMEM((SPC, CH), jnp.int32),
            pltpu.VMEM((SPC, CH), jnp.int32),
            pltpu.VMEM((CH, D), jnp.float32),
            pltpu.VMEM((CH, D), jnp.float32),
            pltpu.VMEM_SHARED((NPAD, D), jnp.float32),
            pltpu.SemaphoreType.DMA,
            pltpu.SemaphoreType.DMA,
            pltpu.SemaphoreType.DMA,
            pltpu.SemaphoreType.DMA,
        ],
    )


def _sc_cnt_body(na, nb, dst0_hbm, dst1_hbm, cnt_hbm, dst_v, cntp_v, sem):
    c = lax.axis_index("c")
    s = lax.axis_index("s")
    wid = s * NC + c

    def zcnt(i, _):
        cntp_v[pl.ds(i * 16, 16)] = jnp.zeros((16,), jnp.float32)
        return 0
    lax.fori_loop(0, CROW * CH // 16, zcnt, 0)

    ones = jnp.ones((16,), jnp.float32)

    def hist(nchunk, dst_hbm):
        def stage(st, _):
            pltpu.sync_copy(dst_hbm.at[s, pl.ds(st * SPC, SPC)], dst_v)

            def chunk(j, _):
                for k in range(CH // 16):
                    iv = dst_v[j, pl.ds(k * 16, 16)]
                    plsc.addupdate_scatter(cntp_v, [iv], ones)
                return 0
            lax.fori_loop(0, SPC, chunk, 0)
            return 0
        lax.fori_loop(0, nchunk // SPC, stage, 0)

    @pl.when(c == 0)
    def _():
        hist(na, dst0_hbm)

    @pl.when(c == 1)
    def _():
        hist(nb, dst1_hbm)

    pltpu.sync_copy(cntp_v, cnt_hbm.at[pl.ds(wid * CROW * CH, CROW * CH)])


def _make_sc_cnt(na, nb):
    mesh = plsc.VectorSubcoreMesh(core_axis_name="c", subcore_axis_name="s")
    return pl.kernel(
        functools.partial(_sc_cnt_body, na, nb),
        out_type=[jax.ShapeDtypeStruct((NC * NS * CROW * CH,), jnp.float32)],
        mesh=mesh,
        compiler_params=pltpu.CompilerParams(needs_layout_passes=False),
        scratch_types=[
            pltpu.VMEM((SPC, CH), jnp.int32),
            pltpu.VMEM((CROW * CH,), jnp.float32),
            pltpu.SemaphoreType.DMA,
        ],
    )


def _cnt_inv_tc(cnt):
    def body(c_ref, o_ref):
        tot = jnp.sum(c_ref[...], axis=0)
        o_ref[...] = 1.0 / jnp.maximum(tot, 1.0)

    return pl.pallas_call(
        body,
        grid=(1,),
        in_specs=[pl.BlockSpec((NC * NS, CROW, CH), lambda i: (0, 0, 0))],
        out_specs=pl.BlockSpec((CROW, CH), lambda i: (0, 0)),
        out_shape=jax.ShapeDtypeStruct((CROW, CH), jnp.float32),
    )(cnt)


def _layer1_tc(acc, inv, xin, W_l, b_l, W_r):
    R = 400
    grid = (N_NODES // R,)

    def body(acc_ref, inv_ref, x_ref, wl_ref, bl_ref, wr_ref, o_ref):
        agg = (acc_ref[0] + acc_ref[1]) * inv_ref[...]
        h = lax.dot_general(agg, wl_ref[...], (((1,), (1,)), ((), ())),
                            preferred_element_type=jnp.float32)
        h = h + lax.dot_general(x_ref[...], wr_ref[...],
                                (((1,), (1,)), ((), ())),
                                preferred_element_type=jnp.float32)
        h = h + bl_ref[...]
        o_ref[...] = jnp.maximum(h, 0.0)

    return pl.pallas_call(
        body,
        grid=grid,
        in_specs=[
            pl.BlockSpec((NC, R, D), lambda i: (0, i, 0)),
            pl.BlockSpec((R, 1), lambda i: (i, 0)),
            pl.BlockSpec((R, D), lambda i: (i, 0)),
            pl.BlockSpec((D, D), lambda i: (0, 0)),
            pl.BlockSpec((1, D), lambda i: (0, 0)),
            pl.BlockSpec((D, D), lambda i: (0, 0)),
        ],
        out_specs=pl.BlockSpec((R, D), lambda i: (i, 0)),
        out_shape=jax.ShapeDtypeStruct((N_NODES, D), jnp.float32),
    )(acc, inv, xin, W_l, b_l, W_r)


def _layer2_tc(acc, inv, hin, W_l, b_l, W_r, W_cls, b_cls):
    R = 400
    grid = (N_NODES // R,)

    def body(acc_ref, inv_ref, h_ref, wl_ref, bl_ref, wr_ref, wc_ref,
             bc_ref, o_ref):
        agg = (acc_ref[0] + acc_ref[1]) * inv_ref[...]
        h = lax.dot_general(agg, wl_ref[...], (((1,), (1,)), ((), ())),
                            preferred_element_type=jnp.float32)
        h = h + lax.dot_general(h_ref[...], wr_ref[...],
                                (((1,), (1,)), ((), ())),
                                preferred_element_type=jnp.float32)
        h = h + bl_ref[...]
        h = jnp.maximum(h, 0.0)
        res = jnp.sum(h * wc_ref[...], axis=1) + bc_ref[0, 0]
        o_ref[...] = res.reshape(R, 1)

    return pl.pallas_call(
        body,
        grid=grid,
        in_specs=[
            pl.BlockSpec((NC, R, D), lambda i: (0, i, 0)),
            pl.BlockSpec((R, 1), lambda i: (i, 0)),
            pl.BlockSpec((R, D), lambda i: (i, 0)),
            pl.BlockSpec((D, D), lambda i: (0, 0)),
            pl.BlockSpec((1, D), lambda i: (0, 0)),
            pl.BlockSpec((D, D), lambda i: (0, 0)),
            pl.BlockSpec((1, D), lambda i: (0, 0)),
            pl.BlockSpec((1, 1), lambda i: (0, 0)),
        ],
        out_specs=pl.BlockSpec((R, 1), lambda i: (i, 0)),
        out_shape=jax.ShapeDtypeStruct((N_NODES, 1), jnp.float32),
    )(acc, inv, hin, W_l, b_l, W_r, W_cls, b_cls)


@jax.jit
def kernel(x, edge_index, W1_l, b1_l, W1_r, W2_l, b2_l, W2_r, W_cls, b_cls):
    n_edges = edge_index.shape[1]
    na, nb = SPLIT
    assert na % SPC == 0 and nb % SPC == 0
    epad = NS * (na + nb) * CH
    assert epad >= n_edges

    src = edge_index[0].astype(jnp.int32)
    dst = edge_index[1].astype(jnp.int32)
    src_f = jnp.concatenate([src, jnp.zeros((epad - n_edges,), jnp.int32)])
    dst_f = jnp.concatenate(
        [dst, jnp.full((epad - n_edges,), N_NODES, jnp.int32)])
    cut = NS * na * CH
    src0 = src_f[:cut].reshape(NS, na, CH)
    dst0 = dst_f[:cut].reshape(NS, na, CH)
    src1 = src_f[cut:].reshape(NS, nb, CH)
    dst1 = dst_f[cut:].reshape(NS, nb, CH)
    zrows = jnp.zeros((CH, D), jnp.float32)

    (cnt,) = _make_sc_cnt(na, nb)(dst0, dst1)
    inv = _cnt_inv_tc(cnt.reshape(NC * NS, CROW, CH))
    inv = inv.reshape(CROW * CH, 1)[:N_NODES]
    (agg1,) = _make_sc_agg(na, nb)(x, src0, dst0, src1, dst1, zrows)
    h1 = _layer1_tc(agg1, inv, x, W1_l, b1_l.reshape(1, D), W1_r)
    (agg2,) = _make_sc_agg(na, nb)(h1, src0, dst0, src1, dst1, zrows)
    out = _layer2_tc(agg2, inv, h1, W2_l, b2_l.reshape(1, D), W2_r,
                     W_cls, b_cls.reshape(1, 1))
    return out.reshape(N_NODES)


# speedup vs baseline: 1.1840x; 1.1840x over previous
import functools
import jax
import jax.numpy as jnp
from jax import lax
from jax.experimental import pallas as pl
from jax.experimental.pallas import tpu as pltpu
from jax.experimental.pallas import tpu_sc as plsc

N_NODES = 10000
D = 128
NC = 2
NS = 16
CH = 128
SPC = 8
NPAD = 10112
RPT = NPAD // NS
CROW = 80
SPLIT = (112, 48)


def _run_stage_loop(nchunk, x_hbm, src_hbm, dst_hbm, acc_sh,
                    src_v, dst_v, rows, sg, ss, s):
    def stage(st, _):
        pltpu.sync_copy(src_hbm.at[s, pl.ds(st * SPC, SPC)], src_v)
        pltpu.sync_copy(dst_hbm.at[s, pl.ds(st * SPC, SPC)], dst_v)
        pltpu.async_copy(x_hbm.at[src_v.at[0]], rows[0], sg[0])
        for j in range(SPC):
            b = j & 1
            if j + 1 < SPC:
                if j >= 1:
                    pltpu.make_async_copy(
                        rows[1 - b], acc_sh.at[dst_v.at[j - 1]],
                        ss[1 - b]).wait()
                pltpu.async_copy(x_hbm.at[src_v.at[j + 1]],
                                 rows[1 - b], sg[1 - b])
            pltpu.make_async_copy(x_hbm.at[src_v.at[j]], rows[b],
                                  sg[b]).wait()
            pltpu.async_copy(rows[b], acc_sh.at[dst_v.at[j]], ss[b],
                             add=True)
        for b in range(2):
            pltpu.make_async_copy(
                rows[b], acc_sh.at[dst_v.at[SPC - 2 + b]], ss[b]).wait()
        return 0
    lax.fori_loop(0, nchunk // SPC, stage, 0)


def _sc_agg_body(na, nb, x_hbm, src0_hbm, dst0_hbm, src1_hbm, dst1_hbm,
                 z_hbm, out_hbm, src_v, dst_v, rows0_v, rows1_v,
                 acc_sh, sg0, sg1, ss0, ss1):
    c = lax.axis_index("c")
    s = lax.axis_index("s")

    nfull = RPT // CH
    rem = RPT % CH
    pltpu.sync_copy(z_hbm, rows0_v)
    for k in range(nfull):
        pltpu.sync_copy(rows0_v, acc_sh.at[pl.ds(s * RPT + k * CH, CH)])
    if rem:
        pltpu.sync_copy(rows0_v.at[pl.ds(0, rem)],
                        acc_sh.at[pl.ds(s * RPT + nfull * CH, rem)])

    plsc.subcore_barrier()

    rows = (rows0_v, rows1_v)
    sg = (sg0, sg1)
    ss = (ss0, ss1)

    @pl.when(c == 0)
    def _():
        _run_stage_loop(na, x_hbm, src0_hbm, dst0_hbm, acc_sh,
                        src_v, dst_v, rows, sg, ss, s)

    @pl.when(c == 1)
    def _():
        _run_stage_loop(nb, x_hbm, src1_hbm, dst1_hbm, acc_sh,
                        src_v, dst_v, rows, sg, ss, s)

    plsc.subcore_barrier()

    for k in range(nfull + (1 if rem else 0)):
        w = CH if k < nfull else rem
        base = s * RPT + k * CH
        pltpu.sync_copy(acc_sh.at[pl.ds(base, w)], rows0_v.at[pl.ds(0, w)])
        pltpu.sync_copy(rows0_v.at[pl.ds(0, w)],
                        out_hbm.at[c, pl.ds(base, w)])


def _make_sc_agg(na, nb):
    mesh = plsc.VectorSubcoreMesh(core_axis_name="c", subcore_axis_name="s")
    return pl.kernel(
        functools.partial(_sc_agg_body, na, nb),
        out_type=[jax.ShapeDtypeStruct((NC, NPAD, D), jnp.float32)],
        mesh=mesh,
        compiler_params=pltpu.CompilerParams(needs_layout_passes=False),
        scratch_types=[
            pltpu.VMEM((SPC, CH), jnp.int32),
            pltpu.VMEM((SPC, CH), jnp.int32),
            pltpu.VMEM((CH, D), jnp.float32),
            pltpu.VMEM((CH, D), jnp.float32),
            pltpu.VMEM_SHARED((NPAD, D), jnp.float32),
            pltpu.SemaphoreType.DMA,
            pltpu.SemaphoreType.DMA,
            pltpu.SemaphoreType.DMA,
            pltpu.SemaphoreType.DMA,
        ],
    )


def _sc_cnt_body(na, nb, dst0_hbm, dst1_hbm, cnt_hbm, dst_v, cntp_v, sem):
    c = lax.axis_index("c")
    s = lax.axis_index("s")
    wid = s * NC + c

    def zcnt(i, _):
        cntp_v[pl.ds(i * 16, 16)] = jnp.zeros((16,), jnp.float32)
        return 0
    lax.fori_loop(0, CROW * CH // 16, zcnt, 0)

    ones = jnp.ones((16,), jnp.float32)

    def hist(nchunk, dst_hbm):
        def stage(st, _):
            pltpu.sync_copy(dst_hbm.at[s, pl.ds(st * SPC, SPC)], dst_v)

            def chunk(j, _):
                for k in range(CH // 16):
                    iv = dst_v[j, pl.ds(k * 16, 16)]
                    plsc.addupdate_scatter(cntp_v, [iv], ones)
                return 0
            lax.fori_loop(0, SPC, chunk, 0)
            return 0
        lax.fori_loop(0, nchunk // SPC, stage, 0)

    @pl.when(c == 0)
    def _():
        hist(na, dst0_hbm)

    @pl.when(c == 1)
    def _():
        hist(nb, dst1_hbm)

    pltpu.sync_copy(cntp_v, cnt_hbm.at[pl.ds(wid * CROW * CH, CROW * CH)])


def _make_sc_cnt(na, nb):
    mesh = plsc.VectorSubcoreMesh(core_axis_name="c", subcore_axis_name="s")
    return pl.kernel(
        functools.partial(_sc_cnt_body, na, nb),
        out_type=[jax.ShapeDtypeStruct((NC * NS * CROW * CH,), jnp.float32)],
        mesh=mesh,
        compiler_params=pltpu.CompilerParams(needs_layout_passes=False),
        scratch_types=[
            pltpu.VMEM((SPC, CH), jnp.int32),
            pltpu.VMEM((CROW * CH,), jnp.float32),
            pltpu.SemaphoreType.DMA,
        ],
    )


def _cnt_inv_tc(cnt):
    def body(c_ref, o_ref):
        tot = jnp.sum(c_ref[...], axis=0)
        o_ref[...] = 1.0 / jnp.maximum(tot, 1.0)

    return pl.pallas_call(
        body,
        grid=(1,),
        in_specs=[pl.BlockSpec((NC * NS, CROW, CH), lambda i: (0, 0, 0))],
        out_specs=pl.BlockSpec((CROW, CH), lambda i: (0, 0)),
        out_shape=jax.ShapeDtypeStruct((CROW, CH), jnp.float32),
    )(cnt)


def _layer1_tc(acc, inv, xin, W_l, b_l, W_r):
    R = 400
    grid = (N_NODES // R,)

    def body(acc_ref, inv_ref, x_ref, wl_ref, bl_ref, wr_ref, o_ref):
        agg = (acc_ref[0] + acc_ref[1]) * inv_ref[...]
        h = lax.dot_general(agg, wl_ref[...], (((1,), (1,)), ((), ())),
                            preferred_element_type=jnp.float32)
        h = h + lax.dot_general(x_ref[...], wr_ref[...],
                                (((1,), (1,)), ((), ())),
                                preferred_element_type=jnp.float32)
        h = h + bl_ref[...]
        o_ref[...] = jnp.maximum(h, 0.0)

    return pl.pallas_call(
        body,
        grid=grid,
        in_specs=[
            pl.BlockSpec((NC, R, D), lambda i: (0, i, 0)),
            pl.BlockSpec((R, 1), lambda i: (i, 0)),
            pl.BlockSpec((R, D), lambda i: (i, 0)),
            pl.BlockSpec((D, D), lambda i: (0, 0)),
            pl.BlockSpec((1, D), lambda i: (0, 0)),
            pl.BlockSpec((D, D), lambda i: (0, 0)),
        ],
        out_specs=pl.BlockSpec((R, D), lambda i: (i, 0)),
        out_shape=jax.ShapeDtypeStruct((N_NODES, D), jnp.float32),
    )(acc, inv, xin, W_l, b_l, W_r)


def _layer2_tc(acc, inv, hin, W_l, b_l, W_r, W_cls, b_cls):
    R = 400
    grid = (N_NODES // R,)

    def body(acc_ref, inv_ref, h_ref, wl_ref, bl_ref, wr_ref, wc_ref,
             bc_ref, o_ref):
        agg = (acc_ref[0] + acc_ref[1]) * inv_ref[...]
        h = lax.dot_general(agg, wl_ref[...], (((1,), (1,)), ((), ())),
                            preferred_element_type=jnp.float32)
        h = h + lax.dot_general(h_ref[...], wr_ref[...],
                                (((1,), (1,)), ((), ())),
                                preferred_element_type=jnp.float32)
        h = h + bl_ref[...]
        h = jnp.maximum(h, 0.0)
        res = jnp.sum(h * wc_ref[...], axis=1) + bc_ref[0, 0]
        o_ref[...] = res.reshape(R, 1)

    return pl.pallas_call(
        body,
        grid=grid,
        in_specs=[
            pl.BlockSpec((NC, R, D), lambda i: (0, i, 0)),
            pl.BlockSpec((R, 1), lambda i: (i, 0)),
            pl.BlockSpec((R, D), lambda i: (i, 0)),
            pl.BlockSpec((D, D), lambda i: (0, 0)),
            pl.BlockSpec((1, D), lambda i: (0, 0)),
            pl.BlockSpec((D, D), lambda i: (0, 0)),
            pl.BlockSpec((1, D), lambda i: (0, 0)),
            pl.BlockSpec((1, 1), lambda i: (0, 0)),
        ],
        out_specs=pl.BlockSpec((R, 1), lambda i: (i, 0)),
        out_shape=jax.ShapeDtypeStruct((N_NODES, 1), jnp.float32),
    )(acc, inv, hin, W_l, b_l, W_r, W_cls, b_cls)


@jax.jit
def kernel(x, edge_index, W1_l, b1_l, W1_r, W2_l, b2_l, W2_r, W_cls, b_cls):
    n_edges = edge_index.shape[1]
    na, nb = SPLIT
    assert na % SPC == 0 and nb % SPC == 0
    epad = NS * (na + nb) * CH
    assert epad >= n_edges

    src = edge_index[0].astype(jnp.int32)
    dst = edge_index[1].astype(jnp.int32)
    src_f = jnp.concatenate([src, jnp.zeros((epad - n_edges,), jnp.int32)])
    dst_f = jnp.concatenate(
        [dst, jnp.full((epad - n_edges,), N_NODES, jnp.int32)])
    cut = NS * na * CH
    src0 = src_f[:cut].reshape(NS, na, CH)
    dst0 = dst_f[:cut].reshape(NS, na, CH)
    src1 = src_f[cut:].reshape(NS, nb, CH)
    dst1 = dst_f[cut:].reshape(NS, nb, CH)
    zrows = jnp.zeros((CH, D), jnp.float32)

    (cnt,) = _make_sc_cnt(na, nb)(dst0, dst1)
    inv = _cnt_inv_tc(cnt.reshape(NC * NS, CROW, CH))
    inv = inv.reshape(CROW * CH, 1)[:N_NODES]
    (agg1,) = _make_sc_agg(na, nb)(x, src0, dst0, src1, dst1, zrows)
    h1 = _layer1_tc(agg1, inv, x, W1_l, b1_l.reshape(1, D), W1_r)
    (agg2,) = _make_sc_agg(na, nb)(h1, src0, dst0, src1, dst1, zrows)
    out = _layer2_tc(agg2, inv, h1, W2_l, b2_l.reshape(1, D), W2_r,
                     W_cls, b_cls.reshape(1, 1))
    return out.reshape(N_NODES)
